# baseline (speedup 1.0000x reference)
import functools

import jax
import jax.numpy as jnp
from jax import lax
from jax.experimental import pallas as pl
from jax.experimental.pallas import tpu as pltpu
from jax.experimental.pallas import tpu_sc as plsc

BATCH = 4096
HIST = 200
NUMV = 1000000
D = 64
DPAD = 128
NC, NS = 2, 16
NW = NC * NS
BBLK = BATCH // NW
NBG = 4
NBS = 2
L = 16
BPITCH = BBLK + 1

_mesh = plsc.VectorSubcoreMesh(core_axis_name="c", subcore_axis_name="s")

_KERNEL_KWARGS = dict(
    mesh=_mesh,
    out_type=jax.ShapeDtypeStruct((HIST, D, BATCH), jnp.float32),
    scratch_types=[
        pltpu.VMEM((HIST, BBLK), jnp.int32),
        pltpu.VMEM((NBG, BBLK, DPAD), jnp.float32),
        pltpu.VMEM((NBS, D, BPITCH), jnp.float32),
        [pltpu.SemaphoreType.DMA] * NBG,
        [pltpu.SemaphoreType.DMA] * NBS,
    ],
    compiler_params=pltpu.CompilerParams(
        use_tc_tiling_on_sc=True, needs_layout_passes=False
    ),
)


def _gather_body(idx_hbm, table_hbm, out_hbm, idx_v, rows_v, tbuf_v,
                 gsems, ssems):
    wid = lax.axis_index("s") * NC + lax.axis_index("c")
    b0 = wid * BBLK
    pltpu.sync_copy(idx_hbm.at[:, pl.ds(b0, BBLK)], idx_v)

    def gather(h, b):
        pltpu.async_copy(table_hbm.at[idx_v.at[h]], rows_v.at[b], gsems[b])

    def writeback(h, s):
        pltpu.async_copy(
            tbuf_v.at[s, :, pl.ds(0, BBLK)],
            out_hbm.at[h, :, pl.ds(b0, BBLK)], ssems[s]
        )

    col_ids = [jnp.arange(L, dtype=jnp.int32) + L * k for k in range(D // L)]

    def transpose(b, s):
        def body_t(t, carry):
            loads = []
            for j in range(8):
                r = t * 8 + j
                rb = jnp.full((L,), r, jnp.int32)
                for k in range(D // L):
                    loads.append((rb, k, rows_v[b, r, pl.ds(L * k, L)]))
            for rb, k, vals in loads:
                plsc.store_scatter(tbuf_v.at[s], [col_ids[k], rb], vals)
            return carry

        lax.fori_loop(0, BBLK // 8, body_t, 0)

    for b in range(NBG):
        gather(b, b)

    def body(t, carry):
        for j in range(NBG):
            h = t * NBG + j
            b = j
            s = j % NBS
            pltpu.make_async_copy(
                table_hbm.at[idx_v.at[h]], rows_v.at[b], gsems[b]
            ).wait()

            @pl.when(h >= NBS)
            def _():
                pltpu.make_async_copy(
                    tbuf_v.at[s, :, pl.ds(0, BBLK)],
                    out_hbm.at[h, :, pl.ds(b0, BBLK)], ssems[s]
                ).wait()

            with jax.named_scope("tp"):
                transpose(b, s)
            writeback(h, s)

            @pl.when(h + NBG < HIST)
            def _():
                gather(h + NBG, b)

        return carry

    lax.fori_loop(0, HIST // NBG, body, 0)

    for s in range(NBS):
        pltpu.make_async_copy(
            tbuf_v.at[s, :, pl.ds(0, BBLK)],
            out_hbm.at[s, :, pl.ds(b0, BBLK)], ssems[s]
        ).wait()


_gather_kernel = pl.kernel(_gather_body, **_KERNEL_KWARGS)


def kernel(indices, table):
    idx_t = indices.T
    table_p = jnp.pad(table, ((0, 0), (0, DPAD - D)))
    out_phys = _gather_kernel(idx_t, table_p)
    return jnp.transpose(out_phys, (2, 0, 1))


# speedup vs baseline: 1.2053x; 1.2053x over previous
import functools

import jax
import jax.numpy as jnp
from jax import lax
from jax.experimental import pallas as pl
from jax.experimental.pallas import tpu as pltpu
from jax.experimental.pallas import tpu_sc as plsc

B_TOTAL = 4096 * 200
D = 64
NC, NS = 2, 16
NW = NC * NS
B_PER_W = B_TOTAL // NW
CHUNK = 256
NBUF = 4
N_CHUNKS = B_PER_W // CHUNK
N_GROUPS = N_CHUNKS // NBUF

_mesh = plsc.VectorSubcoreMesh(core_axis_name="c", subcore_axis_name="s")

_KERNEL_KWARGS = dict(
    mesh=_mesh,
    out_type=jax.ShapeDtypeStruct((B_TOTAL, D), jnp.float32),
    scratch_types=[
        pltpu.VMEM((B_PER_W,), jnp.int32),
        pltpu.VMEM((NBUF, CHUNK, D), jnp.float32),
        [pltpu.SemaphoreType.DMA] * NBUF,
        [pltpu.SemaphoreType.DMA] * NBUF,
    ],
    compiler_params=pltpu.CompilerParams(use_tc_tiling_on_sc=False),
)


def _gather_body(idx_hbm, table_hbm, out_hbm, idx_v, rows_v, gsems, ssems):
    wid = lax.axis_index("s") * NC + lax.axis_index("c")
    base = wid * B_PER_W
    pltpu.sync_copy(idx_hbm.at[pl.ds(base, B_PER_W)], idx_v)

    def gather(chunk, b):
        off = chunk * CHUNK
        pltpu.async_copy(
            table_hbm.at[idx_v.at[pl.ds(off, CHUNK)]], rows_v.at[b], gsems[b]
        )

    def scatter(chunk, b):
        off = chunk * CHUNK
        pltpu.async_copy(
            rows_v.at[b], out_hbm.at[pl.ds(base + off, CHUNK)], ssems[b]
        )

    for b in range(NBUF):
        gather(b, b)

    def group_body(g, carry):
        for b in range(NBUF):
            i = g * NBUF + b
            pltpu.make_async_copy(
                table_hbm.at[idx_v.at[pl.ds(0, CHUNK)]], rows_v.at[b], gsems[b]
            ).wait()
            scatter(i, b)
            pltpu.make_async_copy(
                rows_v.at[b], out_hbm.at[pl.ds(base, CHUNK)], ssems[b]
            ).wait()

            @pl.when(i + NBUF < N_CHUNKS)
            def _():
                gather(i + NBUF, b)

        return carry

    lax.fori_loop(0, N_GROUPS, group_body, 0)


_gather_kernel = pl.kernel(_gather_body, **_KERNEL_KWARGS)


def kernel(indices, table):
    flat = indices.reshape(-1)
    out = _gather_kernel(flat, table)
    return out.reshape(indices.shape + (D,))
